# Initial kernel scaffold; baseline (speedup 1.0000x reference)
import jax
import jax.numpy as jnp
from jax.experimental import pallas as pl


def kernel(logits, fnode, fmess, agraph, bgraph, mask, W_z, b_z, W_r, U_r, b_Ur, W_h, b_h, W_f, b_f, W_i, b_i, W_m1, b_m1, W_m2, b_m2):
    raise NotImplementedError("write your pallas kernel here")


# speedup vs baseline: 1.8112x; 1.8112x over previous
import functools

import jax
import jax.numpy as jnp
from jax import lax
from jax.experimental import pallas as pl
from jax.experimental.pallas import tpu as pltpu, tpu_sc as plsc

N = 10000
E = 160000
MAX_NB = 6
H = 128
FD = 16

NC = 2
NS = 16
NW = NC * NS

_MESH = plsc.VectorSubcoreMesh(core_axis_name="c", subcore_axis_name="s")


def _wid():
    return lax.axis_index("s") * NC + lax.axis_index("c")


def _sc_gather(table, idx):
    rows = idx.shape[0]
    rpw = rows // NW
    ch = 200
    nch = rpw // ch

    @functools.partial(
        pl.kernel, mesh=_MESH,
        out_type=jax.ShapeDtypeStruct((rows, H), jnp.float32),
        scratch_types=[
            pltpu.VMEM((ch,), jnp.int32),
            pltpu.VMEM((ch, H), jnp.float32),
            pltpu.SemaphoreType.DMA,
        ],
    )
    def k(table_hbm, idx_hbm, out_hbm, idx_v, rows_v, sem):
        base = _wid() * rpw

        def chunk(ci, carry):
            off = base + ci * ch
            pltpu.sync_copy(idx_hbm.at[pl.ds(off, ch)], idx_v)
            pltpu.async_copy(table_hbm.at[idx_v], rows_v, sem).wait()
            pltpu.sync_copy(rows_v, out_hbm.at[pl.ds(off, ch)])
            return carry

        lax.fori_loop(0, nch, chunk, 0)

    return k(table, idx)


def _sc_neigh(G, bidx, A_r):
    epw = E // NW
    C = 40
    nch = epw // C

    @functools.partial(
        pl.kernel, mesh=_MESH,
        out_type=(jax.ShapeDtypeStruct((E, H), jnp.float32),
                  jax.ShapeDtypeStruct((E, H), jnp.float32)),
        scratch_types=[
            pltpu.VMEM((C * MAX_NB,), jnp.int32),
            pltpu.VMEM((C * MAX_NB, 2 * H), jnp.float32),
            pltpu.VMEM((C, H), jnp.float32),
            pltpu.VMEM((C, H), jnp.float32),
            pltpu.VMEM((C, H), jnp.float32),
            pltpu.SemaphoreType.DMA,
        ],
    )
    def k(g_hbm, bidx_hbm, ar_hbm, sumh_hbm, sumg_hbm,
          idx_v, rows_v, ar_v, sh_v, sg_v, sem):
        base = _wid() * epw

        def chunk(ci, carry):
            eb = base + ci * C
            pltpu.sync_copy(bidx_hbm.at[pl.ds(eb * MAX_NB, C * MAX_NB)], idx_v)
            cp = pltpu.async_copy(g_hbm.at[idx_v], rows_v, sem)
            pltpu.sync_copy(ar_hbm.at[pl.ds(eb, C)], ar_v)
            cp.wait()

            def edge(e, carry2):
                for kk in range(H // 16):
                    sl = pl.ds(kk * 16, 16)
                    sl2 = pl.ds(H + kk * 16, 16)
                    ar = ar_v[e, sl]
                    s = jnp.zeros((16,), jnp.float32)
                    g = jnp.zeros((16,), jnp.float32)
                    for j in range(MAX_NB):
                        hv = rows_v[e * MAX_NB + j, sl]
                        hu = rows_v[e * MAX_NB + j, sl2]
                        s = s + hv
                        sig = 1.0 / (1.0 + jnp.exp(-(ar + hu)))
                        g = g + sig * hv
                    sh_v[e, sl] = s
                    sg_v[e, sl] = g
                return carry2

            lax.fori_loop(0, C, edge, 0)
            pltpu.sync_copy(sh_v, sumh_hbm.at[pl.ds(eb, C)])
            pltpu.sync_copy(sg_v, sumg_hbm.at[pl.ds(eb, C)])
            return carry

        lax.fori_loop(0, nch, chunk, 0)

    return k(G, bidx, A_r)


_NPAD = 10240


def _sc_nodeagg(h, aidx):
    npw = _NPAD // NW
    C = 64
    nch = npw // C

    @functools.partial(
        pl.kernel, mesh=_MESH,
        out_type=jax.ShapeDtypeStruct((_NPAD, H), jnp.float32),
        scratch_types=[
            pltpu.VMEM((C * MAX_NB,), jnp.int32),
            pltpu.VMEM((C * MAX_NB, H), jnp.float32),
            pltpu.VMEM((C, H), jnp.float32),
            pltpu.SemaphoreType.DMA,
        ],
    )
    def k(h_hbm, aidx_hbm, out_hbm, idx_v, rows_v, acc_v, sem):
        base = _wid() * npw

        def chunk(ci, carry):
            nb = base + ci * C
            pltpu.sync_copy(aidx_hbm.at[pl.ds(nb * MAX_NB, C * MAX_NB)], idx_v)
            pltpu.async_copy(h_hbm.at[idx_v], rows_v, sem).wait()

            def node(e, carry2):
                for kk in range(H // 16):
                    sl = pl.ds(kk * 16, 16)
                    s = jnp.zeros((16,), jnp.float32)
                    for j in range(MAX_NB):
                        s = s + rows_v[e * MAX_NB + j, sl]
                    acc_v[e, sl] = s
                return carry2

            lax.fori_loop(0, C, node, 0)
            pltpu.sync_copy(acc_v, out_hbm.at[pl.ds(nb, C)])
            return carry

        lax.fori_loop(0, nch, chunk, 0)

    return k(h, aidx)


_EB = 2000
_NB_ = 400


def _row_keep(pid, nrows):
    r = lax.broadcasted_iota(jnp.int32, (nrows, 1), 0) + pid * nrows
    return (r != 0).astype(jnp.float32)


def _tc_prep(fsrc, ef, Wz1a, Wz1b, bz, Wra, Wrb, Wh1a, Wh1b, bh, Ur, bUr):
    def body(fs, e, wza, wzb, bz_, wra, wrb, wha, whb, bh_, ur, bur,
             az_o, ar_o, ah_o, g_o):
        fs_ = fs[...]
        e_ = e[...]
        az = fs_ @ wza[...] + e_ @ wzb[...] + bz_[...]
        ar = fs_ @ wra[...] + e_ @ wrb[...]
        ah = fs_ @ wha[...] + e_ @ whb[...] + bh_[...]
        h = jax.nn.sigmoid(az) * jnp.tanh(ah)
        h = h * _row_keep(pl.program_id(0), fs_.shape[0])
        hu = h @ ur[...] + bur[...]
        az_o[...] = az
        ar_o[...] = ar
        ah_o[...] = ah
        g_o[...] = jnp.concatenate([h, hu], axis=1)

    grid = E // _EB
    row = lambda i: (i, 0)
    full = lambda i: (0, 0)
    return pl.pallas_call(
        body,
        grid=(grid,),
        in_specs=[
            pl.BlockSpec((_EB, H), row),
            pl.BlockSpec((_EB, FD), row),
            pl.BlockSpec((H, H), full),
            pl.BlockSpec((FD, H), full),
            pl.BlockSpec((1, H), full),
            pl.BlockSpec((H, H), full),
            pl.BlockSpec((FD, H), full),
            pl.BlockSpec((H, H), full),
            pl.BlockSpec((FD, H), full),
            pl.BlockSpec((1, H), full),
            pl.BlockSpec((H, H), full),
            pl.BlockSpec((1, H), full),
        ],
        out_specs=[
            pl.BlockSpec((_EB, H), row),
            pl.BlockSpec((_EB, H), row),
            pl.BlockSpec((_EB, H), row),
            pl.BlockSpec((_EB, 2 * H), row),
        ],
        out_shape=[
            jax.ShapeDtypeStruct((E, H), jnp.float32),
            jax.ShapeDtypeStruct((E, H), jnp.float32),
            jax.ShapeDtypeStruct((E, H), jnp.float32),
            jax.ShapeDtypeStruct((E, 2 * H), jnp.float32),
        ],
    )(fsrc, ef, Wz1a, Wz1b, bz, Wra, Wrb, Wh1a, Wh1b, bh, Ur, bUr)


def _tc_update(A_z, A_h, sumh, sumg, Wz2, Wh2, Ur, bUr, last):
    def body(az, ah, sh, sg, wz2, wh2, *rest):
        if last:
            (h_o,) = rest
        else:
            ur, bur, g_o = rest
        sh_ = sh[...]
        sg_ = sg[...]
        z = jax.nn.sigmoid(az[...] + sh_ @ wz2[...])
        p = jnp.tanh(ah[...] + sg_ @ wh2[...])
        h = (1.0 - z) * sh_ + z * p
        h = h * _row_keep(pl.program_id(0), sh_.shape[0])
        if last:
            h_o[...] = h
        else:
            hu = h @ ur[...] + bur[...]
            g_o[...] = jnp.concatenate([h, hu], axis=1)

    grid = E // _EB
    row = lambda i: (i, 0)
    full = lambda i: (0, 0)
    in_specs = [
        pl.BlockSpec((_EB, H), row),
        pl.BlockSpec((_EB, H), row),
        pl.BlockSpec((_EB, H), row),
        pl.BlockSpec((_EB, H), row),
        pl.BlockSpec((H, H), full),
        pl.BlockSpec((H, H), full),
    ]
    args = [A_z, A_h, sumh, sumg, Wz2, Wh2]
    if last:
        out_specs = pl.BlockSpec((_EB, H), row)
        out_shape = jax.ShapeDtypeStruct((E, H), jnp.float32)
    else:
        in_specs += [pl.BlockSpec((H, H), full), pl.BlockSpec((1, H), full)]
        args += [Ur, bUr]
        out_specs = pl.BlockSpec((_EB, 2 * H), row)
        out_shape = jax.ShapeDtypeStruct((E, 2 * H), jnp.float32)
    return pl.pallas_call(
        body, grid=(grid,), in_specs=in_specs,
        out_specs=out_specs, out_shape=out_shape,
    )(*args)


def _tc_final(fnode, nei, logits, mask,
              Wfa, Wfb, bf, Wia, Wib, bi, Wm1a, Wm1b, bm1, Wm2, bm2):
    def body(fn, ne, lo, mk, wfa, wfb, bf_, wia, wib, bi_,
             wm1a, wm1b, bm1_, wm2, bm2_, out):
        fn_ = fn[...]
        ne_ = ne[...]
        f = jax.nn.sigmoid(fn_ @ wfa[...] + ne_ @ wfb[...] + bf_[...])
        i = jax.nn.sigmoid(fn_ @ wia[...] + ne_ @ wib[...] + bi_[...])
        m1 = jax.nn.relu(fn_ @ wm1a[...] + ne_ @ wm1b[...] + bm1_[...])
        mt = m1 @ wm2[...] + bm2_[...]
        out[...] = (f * lo[...] + i * mt) * mk[...]

    grid = N // _NB_
    row = lambda i: (i, 0)
    full = lambda i: (0, 0)
    return pl.pallas_call(
        body,
        grid=(grid,),
        in_specs=[
            pl.BlockSpec((_NB_, H), row),
            pl.BlockSpec((_NB_, H), row),
            pl.BlockSpec((_NB_, H), row),
            pl.BlockSpec((_NB_, 1), row),
            pl.BlockSpec((H, H), full),
            pl.BlockSpec((H, H), full),
            pl.BlockSpec((1, H), full),
            pl.BlockSpec((H, H), full),
            pl.BlockSpec((H, H), full),
            pl.BlockSpec((1, H), full),
            pl.BlockSpec((H, H), full),
            pl.BlockSpec((H, H), full),
            pl.BlockSpec((1, H), full),
            pl.BlockSpec((H, H), full),
            pl.BlockSpec((1, H), full),
        ],
        out_specs=pl.BlockSpec((_NB_, H), row),
        out_shape=jax.ShapeDtypeStruct((N, H), jnp.float32),
    )(fnode, nei, logits, mask, Wfa, Wfb, bf, Wia, Wib, bi,
      Wm1a, Wm1b, bm1, Wm2, bm2)


def kernel(logits, fnode, fmess, agraph, bgraph, mask,
           W_z, b_z, W_r, U_r, b_Ur, W_h, b_h,
           W_f, b_f, W_i, b_i, W_m1, b_m1, W_m2, b_m2):
    src_idx = fmess[:, 0].astype(jnp.int32)
    ef = fmess[:, 2:]
    bidx = bgraph.astype(jnp.int32).reshape(-1)
    aidx = jnp.pad(agraph.astype(jnp.int32).reshape(-1),
                   (0, (_NPAD - N) * MAX_NB))

    bz = b_z.reshape(1, H)
    bh = b_h.reshape(1, H)
    bur = b_Ur.reshape(1, H)

    fsrc = _sc_gather(fnode, src_idx)
    A_z, A_r, A_h, G = _tc_prep(
        fsrc, ef,
        W_z[:H], W_z[H:H + FD], bz,
        W_r[:H], W_r[H:H + FD],
        W_h[:H], W_h[H:H + FD], bh,
        U_r, bur)
    Wz2 = W_z[H + FD:]
    Wh2 = W_h[H + FD:]

    sumh, sumg = _sc_neigh(G, bidx, A_r)
    G = _tc_update(A_z, A_h, sumh, sumg, Wz2, Wh2, U_r, bur, last=False)
    sumh, sumg = _sc_neigh(G, bidx, A_r)
    h = _tc_update(A_z, A_h, sumh, sumg, Wz2, Wh2, None, None, last=True)

    nei = _sc_nodeagg(h, aidx)[:N]
    return _tc_final(
        fnode, nei, logits, mask,
        W_f[:H], W_f[H:], b_f.reshape(1, H),
        W_i[:H], W_i[H:], b_i.reshape(1, H),
        W_m1[:H], W_m1[H:], b_m1.reshape(1, H),
        W_m2, b_m2.reshape(1, H))


# speedup vs baseline: 5.7611x; 3.1808x over previous
import functools

import jax
import jax.numpy as jnp
from jax import lax
from jax.experimental import pallas as pl
from jax.experimental.pallas import tpu as pltpu, tpu_sc as plsc

N = 10000
E = 160000
MAX_NB = 6
H = 128
FD = 16

NC = 2
NS = 16
NW = NC * NS

_MESH = plsc.VectorSubcoreMesh(core_axis_name="c", subcore_axis_name="s")


def _wid():
    return lax.axis_index("s") * NC + lax.axis_index("c")


def _sc_gather(table, idx):
    rows = idx.shape[0]
    rpw = rows // NW
    ch = 200
    nch = rpw // ch

    @functools.partial(
        pl.kernel, mesh=_MESH,
        out_type=jax.ShapeDtypeStruct((rows, H), jnp.float32),
        scratch_types=[
            pltpu.VMEM((ch,), jnp.int32),
            pltpu.VMEM((ch, H), jnp.float32),
            pltpu.SemaphoreType.DMA,
        ],
    )
    def k(table_hbm, idx_hbm, out_hbm, idx_v, rows_v, sem):
        base = _wid() * rpw

        def chunk(ci, carry):
            off = base + ci * ch
            pltpu.sync_copy(idx_hbm.at[pl.ds(off, ch)], idx_v)
            pltpu.async_copy(table_hbm.at[idx_v], rows_v, sem).wait()
            pltpu.sync_copy(rows_v, out_hbm.at[pl.ds(off, ch)])
            return carry

        lax.fori_loop(0, nch, chunk, 0)

    return k(table, idx)


def _sc_neigh(G, bidx, A_r):
    epw = E // NW
    U = 40
    HF = U // 2
    nu = epw // U

    @functools.partial(
        pl.kernel, mesh=_MESH,
        out_type=(jax.ShapeDtypeStruct((E, H), jnp.float32),
                  jax.ShapeDtypeStruct((E, H), jnp.float32)),
        scratch_types=[
            pltpu.VMEM((2, HF * MAX_NB), jnp.int32),
            pltpu.VMEM((2, HF * MAX_NB, 2 * H), jnp.float32),
            pltpu.VMEM((U, H), jnp.float32),
            pltpu.VMEM((U, H), jnp.float32),
            pltpu.VMEM((U, H), jnp.float32),
            pltpu.SemaphoreType.DMA,
            pltpu.SemaphoreType.DMA,
        ],
    )
    def k(g_hbm, bidx_hbm, ar_hbm, sumh_hbm, sumg_hbm,
          idx_v, rows_v, ar_v, sh_v, sg_v, sem0, sem1):
        base = _wid() * epw
        sems = (sem0, sem1)

        def issue(p, s):
            off6 = (base + s * HF) * MAX_NB
            pltpu.sync_copy(bidx_hbm.at[pl.ds(off6, HF * MAX_NB)], idx_v.at[p])
            pltpu.async_copy(g_hbm.at[idx_v.at[p]], rows_v.at[p], sems[p])

        def wait(p):
            pltpu.make_async_copy(g_hbm.at[idx_v.at[p]], rows_v.at[p],
                                  sems[p]).wait()

        def compute(p, t):
            @functools.partial(plsc.parallel_loop, 0, HF, unroll=2)
            def edge(e):
                eo = t * HF + e
                for kk in range(H // 16):
                    sl = pl.ds(kk * 16, 16)
                    sl2 = pl.ds(H + kk * 16, 16)
                    nar = -ar_v[eo, sl]
                    s = jnp.zeros((16,), jnp.float32)
                    g = jnp.zeros((16,), jnp.float32)
                    for j in range(MAX_NB):
                        hv = rows_v[p, e * MAX_NB + j, sl]
                        hu = rows_v[p, e * MAX_NB + j, sl2]
                        s = s + hv
                        d = 1.0 + jnp.exp(nar - hu)
                        g = g + hv / d
                    sh_v[eo, sl] = s
                    sg_v[eo, sl] = g

        issue(0, 0)

        def unit(u, carry):
            off = base + u * U
            pltpu.sync_copy(ar_hbm.at[pl.ds(off, U)], ar_v)
            issue(1, 2 * u + 1)
            wait(0)
            compute(0, 0)

            @pl.when(u < nu - 1)
            def _():
                issue(0, 2 * u + 2)

            wait(1)
            compute(1, 1)
            pltpu.sync_copy(sh_v, sumh_hbm.at[pl.ds(off, U)])
            pltpu.sync_copy(sg_v, sumg_hbm.at[pl.ds(off, U)])
            return carry

        lax.fori_loop(0, nu, unit, 0)

    return k(G, bidx, A_r)


_NPAD = 10240


def _sc_nodeagg(h, aidx):
    npw = _NPAD // NW
    C = 64
    nch = npw // C

    @functools.partial(
        pl.kernel, mesh=_MESH,
        out_type=jax.ShapeDtypeStruct((_NPAD, H), jnp.float32),
        scratch_types=[
            pltpu.VMEM((C * MAX_NB,), jnp.int32),
            pltpu.VMEM((C * MAX_NB, H), jnp.float32),
            pltpu.VMEM((C, H), jnp.float32),
            pltpu.SemaphoreType.DMA,
        ],
    )
    def k(h_hbm, aidx_hbm, out_hbm, idx_v, rows_v, acc_v, sem):
        base = _wid() * npw

        def chunk(ci, carry):
            nb = base + ci * C
            pltpu.sync_copy(aidx_hbm.at[pl.ds(nb * MAX_NB, C * MAX_NB)], idx_v)
            pltpu.async_copy(h_hbm.at[idx_v], rows_v, sem).wait()

            def node(e, carry2):
                for kk in range(H // 16):
                    sl = pl.ds(kk * 16, 16)
                    s = jnp.zeros((16,), jnp.float32)
                    for j in range(MAX_NB):
                        s = s + rows_v[e * MAX_NB + j, sl]
                    acc_v[e, sl] = s
                return carry2

            lax.fori_loop(0, C, node, 0)
            pltpu.sync_copy(acc_v, out_hbm.at[pl.ds(nb, C)])
            return carry

        lax.fori_loop(0, nch, chunk, 0)

    return k(h, aidx)


_EB = 2000
_NB_ = 400


def _row_keep(pid, nrows):
    r = lax.broadcasted_iota(jnp.int32, (nrows, 1), 0) + pid * nrows
    return (r != 0).astype(jnp.float32)


def _tc_prep(fsrc, ef, Wz1a, Wz1b, bz, Wra, Wrb, Wh1a, Wh1b, bh, Ur, bUr):
    def body(fs, e, wza, wzb, bz_, wra, wrb, wha, whb, bh_, ur, bur,
             az_o, ar_o, ah_o, g_o):
        fs_ = fs[...]
        e_ = e[...]
        az = fs_ @ wza[...] + e_ @ wzb[...] + bz_[...]
        ar = fs_ @ wra[...] + e_ @ wrb[...]
        ah = fs_ @ wha[...] + e_ @ whb[...] + bh_[...]
        h = jax.nn.sigmoid(az) * jnp.tanh(ah)
        h = h * _row_keep(pl.program_id(0), fs_.shape[0])
        hu = h @ ur[...] + bur[...]
        az_o[...] = az
        ar_o[...] = ar
        ah_o[...] = ah
        g_o[...] = jnp.concatenate([h, hu], axis=1)

    grid = E // _EB
    row = lambda i: (i, 0)
    full = lambda i: (0, 0)
    return pl.pallas_call(
        body,
        grid=(grid,),
        in_specs=[
            pl.BlockSpec((_EB, H), row),
            pl.BlockSpec((_EB, FD), row),
            pl.BlockSpec((H, H), full),
            pl.BlockSpec((FD, H), full),
            pl.BlockSpec((1, H), full),
            pl.BlockSpec((H, H), full),
            pl.BlockSpec((FD, H), full),
            pl.BlockSpec((H, H), full),
            pl.BlockSpec((FD, H), full),
            pl.BlockSpec((1, H), full),
            pl.BlockSpec((H, H), full),
            pl.BlockSpec((1, H), full),
        ],
        out_specs=[
            pl.BlockSpec((_EB, H), row),
            pl.BlockSpec((_EB, H), row),
            pl.BlockSpec((_EB, H), row),
            pl.BlockSpec((_EB, 2 * H), row),
        ],
        out_shape=[
            jax.ShapeDtypeStruct((E, H), jnp.float32),
            jax.ShapeDtypeStruct((E, H), jnp.float32),
            jax.ShapeDtypeStruct((E, H), jnp.float32),
            jax.ShapeDtypeStruct((E, 2 * H), jnp.float32),
        ],
    )(fsrc, ef, Wz1a, Wz1b, bz, Wra, Wrb, Wh1a, Wh1b, bh, Ur, bUr)


def _tc_update(A_z, A_h, sumh, sumg, Wz2, Wh2, Ur, bUr, last):
    def body(az, ah, sh, sg, wz2, wh2, *rest):
        if last:
            (h_o,) = rest
        else:
            ur, bur, g_o = rest
        sh_ = sh[...]
        sg_ = sg[...]
        z = jax.nn.sigmoid(az[...] + sh_ @ wz2[...])
        p = jnp.tanh(ah[...] + sg_ @ wh2[...])
        h = (1.0 - z) * sh_ + z * p
        h = h * _row_keep(pl.program_id(0), sh_.shape[0])
        if last:
            h_o[...] = h
        else:
            hu = h @ ur[...] + bur[...]
            g_o[...] = jnp.concatenate([h, hu], axis=1)

    grid = E // _EB
    row = lambda i: (i, 0)
    full = lambda i: (0, 0)
    in_specs = [
        pl.BlockSpec((_EB, H), row),
        pl.BlockSpec((_EB, H), row),
        pl.BlockSpec((_EB, H), row),
        pl.BlockSpec((_EB, H), row),
        pl.BlockSpec((H, H), full),
        pl.BlockSpec((H, H), full),
    ]
    args = [A_z, A_h, sumh, sumg, Wz2, Wh2]
    if last:
        out_specs = pl.BlockSpec((_EB, H), row)
        out_shape = jax.ShapeDtypeStruct((E, H), jnp.float32)
    else:
        in_specs += [pl.BlockSpec((H, H), full), pl.BlockSpec((1, H), full)]
        args += [Ur, bUr]
        out_specs = pl.BlockSpec((_EB, 2 * H), row)
        out_shape = jax.ShapeDtypeStruct((E, 2 * H), jnp.float32)
    return pl.pallas_call(
        body, grid=(grid,), in_specs=in_specs,
        out_specs=out_specs, out_shape=out_shape,
    )(*args)


def _tc_final(fnode, nei, logits, mask,
              Wfa, Wfb, bf, Wia, Wib, bi, Wm1a, Wm1b, bm1, Wm2, bm2):
    def body(fn, ne, lo, mk, wfa, wfb, bf_, wia, wib, bi_,
             wm1a, wm1b, bm1_, wm2, bm2_, out):
        fn_ = fn[...]
        ne_ = ne[...]
        f = jax.nn.sigmoid(fn_ @ wfa[...] + ne_ @ wfb[...] + bf_[...])
        i = jax.nn.sigmoid(fn_ @ wia[...] + ne_ @ wib[...] + bi_[...])
        m1 = jax.nn.relu(fn_ @ wm1a[...] + ne_ @ wm1b[...] + bm1_[...])
        mt = m1 @ wm2[...] + bm2_[...]
        out[...] = (f * lo[...] + i * mt) * mk[...]

    grid = N // _NB_
    row = lambda i: (i, 0)
    full = lambda i: (0, 0)
    return pl.pallas_call(
        body,
        grid=(grid,),
        in_specs=[
            pl.BlockSpec((_NB_, H), row),
            pl.BlockSpec((_NB_, H), row),
            pl.BlockSpec((_NB_, H), row),
            pl.BlockSpec((_NB_, 1), row),
            pl.BlockSpec((H, H), full),
            pl.BlockSpec((H, H), full),
            pl.BlockSpec((1, H), full),
            pl.BlockSpec((H, H), full),
            pl.BlockSpec((H, H), full),
            pl.BlockSpec((1, H), full),
            pl.BlockSpec((H, H), full),
            pl.BlockSpec((H, H), full),
            pl.BlockSpec((1, H), full),
            pl.BlockSpec((H, H), full),
            pl.BlockSpec((1, H), full),
        ],
        out_specs=pl.BlockSpec((_NB_, H), row),
        out_shape=jax.ShapeDtypeStruct((N, H), jnp.float32),
    )(fnode, nei, logits, mask, Wfa, Wfb, bf, Wia, Wib, bi,
      Wm1a, Wm1b, bm1, Wm2, bm2)


def kernel(logits, fnode, fmess, agraph, bgraph, mask,
           W_z, b_z, W_r, U_r, b_Ur, W_h, b_h,
           W_f, b_f, W_i, b_i, W_m1, b_m1, W_m2, b_m2):
    src_idx = fmess[:, 0].astype(jnp.int32)
    ef = fmess[:, 2:]
    bidx = bgraph.astype(jnp.int32).reshape(-1)
    aidx = jnp.pad(agraph.astype(jnp.int32).reshape(-1),
                   (0, (_NPAD - N) * MAX_NB))

    bz = b_z.reshape(1, H)
    bh = b_h.reshape(1, H)
    bur = b_Ur.reshape(1, H)

    fsrc = _sc_gather(fnode, src_idx)
    A_z, A_r, A_h, G = _tc_prep(
        fsrc, ef,
        W_z[:H], W_z[H:H + FD], bz,
        W_r[:H], W_r[H:H + FD],
        W_h[:H], W_h[H:H + FD], bh,
        U_r, bur)
    Wz2 = W_z[H + FD:]
    Wh2 = W_h[H + FD:]

    sumh, sumg = _sc_neigh(G, bidx, A_r)
    G = _tc_update(A_z, A_h, sumh, sumg, Wz2, Wh2, U_r, bur, last=False)
    sumh, sumg = _sc_neigh(G, bidx, A_r)
    h = _tc_update(A_z, A_h, sumh, sumg, Wz2, Wh2, None, None, last=True)

    nei = _sc_nodeagg(h, aidx)[:N]
    return _tc_final(
        fnode, nei, logits, mask,
        W_f[:H], W_f[H:], b_f.reshape(1, H),
        W_i[:H], W_i[H:], b_i.reshape(1, H),
        W_m1[:H], W_m1[H:], b_m1.reshape(1, H),
        W_m2, b_m2.reshape(1, H))
